# baseline (speedup 1.0000x reference)
import functools
import jax
import jax.numpy as jnp
from jax import lax
from jax.experimental import pallas as pl
from jax.experimental.pallas import tpu as pltpu
from jax.experimental.pallas import tpu_sc as plsc

_T5 = 297
_TILE = 152
_WROWS = 312
_NK = 320
_PC = 512
_CW = 128


def _dot(a, b):
    return jax.lax.dot_general(
        a.astype(jnp.bfloat16), b.astype(jnp.bfloat16),
        (((1,), (0,)), ((), ())), preferred_element_type=jnp.float32)


def _phase_conv(xs, taps, bias, s_out):
    p_in = len(xs)
    outs = []
    for q in range(p_in // 2):
        acc = None
        for k in range(taps.shape[0]):
            u = 2 * q + k
            src = xs[u % p_in]
            j = u // p_in
            term = _dot(jax.lax.slice(src, (j, 0), (j + s_out, src.shape[1])),
                        taps[k])
            acc = term if acc is None else acc + term
        outs.append(jnp.maximum(acc + bias, 0.0))
    return outs


def _body(wav_ref, w0_ref, b0_ref, w1_ref, b1_ref, w2_ref, b2_ref,
          w3_ref, b3_ref, w4_ref, b4_ref, cbt_ref, d0_ref, d1_ref):
    t = pl.program_id(1)
    wt = wav_ref[0, pl.ds(_TILE * t, 158), :]
    b0 = b0_ref[...]
    w0 = w0_ref[...]
    x = []
    for p in range(16):
        if p < 15:
            v = _dot(wt[:157, 5 * p:5 * p + 10], w0)
        else:
            pat = jnp.concatenate([wt[:157, 75:80], wt[1:158, 0:5]], axis=1)
            v = _dot(pat, w0)
        x.append(jnp.maximum(v + b0, 0.0))
    x = _phase_conv(x, w1_ref[...], b1_ref[...], 156)
    x = _phase_conv(x, w2_ref[...], b2_ref[...], 155)
    x = _phase_conv(x, w3_ref[...], b3_ref[...], 154)
    x = _phase_conv(x, w4_ref[...], b4_ref[...], _TILE + 1)
    z = x[0][:_TILE]
    for g, out_ref in ((0, d0_ref), (1, d1_ref)):
        zg = z[:, g * 256:(g + 1) * 256]
        cbt = cbt_ref[g]
        zn = jnp.sum(zg * zg, axis=1, keepdims=True)
        en = jnp.sum(cbt * cbt, axis=0, keepdims=True)
        d = (zn - 2.0 * _dot(zg, cbt)) + en
        out_ref[0, :, :_TILE] = d.T


def _tc_distances(wav, ops):
    B = wav.shape[0]
    full = lambda a: pl.BlockSpec(a.shape, lambda b, t: (0,) * a.ndim)
    in_specs = [pl.BlockSpec((1, _WROWS, 80), lambda b, t: (b, 0, 0))]
    in_specs += [full(a) for a in ops]
    return pl.pallas_call(
        _body,
        grid=(B, 2),
        in_specs=in_specs,
        out_specs=[pl.BlockSpec((1, _NK, _PC // 2),
                                lambda b, t: (b, 0, t))] * 2,
        out_shape=[jax.ShapeDtypeStruct((B, _NK, _PC), jnp.float32)] * 2,
    )(wav, *ops)


def _sc_argmin(dt):
    mesh = plsc.VectorSubcoreMesh(core_axis_name="c", subcore_axis_name="s")

    @functools.partial(
        pl.kernel,
        out_type=jax.ShapeDtypeStruct((4, _PC), jnp.int32),
        mesh=mesh,
        scratch_types=[
            pltpu.VMEM((_NK, _CW), jnp.float32),
            pltpu.VMEM((_CW,), jnp.int32),
            pltpu.SemaphoreType.DMA,
        ],
    )
    def body(dt_hbm, out_hbm, chunk_v, out_v, sem):
        wid = (lax.axis_index("s") * 2 + lax.axis_index("c")) % 16
        m = wid // 4
        c0 = (wid % 4) * _CW
        pltpu.sync_copy(dt_hbm.at[m, :, pl.ds(c0, _CW)], chunk_v)
        for grp in range(_CW // 16):
            def step(k, carry):
                vmin, vidx = carry
                v = chunk_v[k, pl.ds(grp * 16, 16)]
                pred = v < vmin
                kv = jnp.broadcast_to(k, (16,)).astype(jnp.int32)
                return (jnp.where(pred, v, vmin), jnp.where(pred, kv, vidx))
            vmin0 = jnp.full((16,), jnp.inf, jnp.float32)
            vidx0 = jnp.zeros((16,), jnp.int32)
            _, vidx = lax.fori_loop(0, _NK, step, (vmin0, vidx0))
            out_v[pl.ds(grp * 16, 16)] = vidx
        pltpu.sync_copy(out_v, out_hbm.at[m, pl.ds(c0, _CW)])

    return body(dt)


def kernel(wav_input, W0, b0, W1, b1, W2, b2, W3, b3, W4, b4, codebook):
    B = wav_input.shape[0]
    wav = jnp.pad(wav_input, ((0, 0), (0, _WROWS * 80 - 24000)))
    wav = wav.reshape(B, _WROWS, 80)
    w0 = W0.reshape(512, 10).T
    w1 = jnp.transpose(W1, (2, 1, 0))
    w2 = jnp.transpose(W2, (2, 1, 0))
    w3 = jnp.transpose(W3, (2, 1, 0))
    w4 = jnp.transpose(W4, (2, 1, 0))
    cbt = jnp.transpose(codebook, (1, 2, 0))
    biases = [b.reshape(1, 512) for b in (b0, b1, b2, b3, b4)]
    ops = [w0, biases[0], w1, biases[1], w2, biases[2],
           w3, biases[3], w4, biases[4], cbt]

    d0t, d1t = _tc_distances(wav, ops)
    dt = jnp.concatenate([d0t, d1t], axis=0)
    isc = _sc_argmin(dt)
    iv = jnp.concatenate([isc[:, :_TILE], isc[:, 256:256 + (_T5 - _TILE)]],
                         axis=1)
    i0 = iv[:B]
    i1 = iv[B:]
    idx = jnp.stack([i0, i1], axis=-1)
    return idx.reshape(B, _T5 * 2)
